# baseline (speedup 1.0000x reference)
import jax
import jax.numpy as jnp
from jax.experimental import pallas as pl
from jax.experimental.pallas import tpu as pltpu

_B, _C, _N = 4, 32, 4096
_TM = 512
_J = _N // _TM


def _chamfer_body(inp_ref, tgt_ref, maskx_ref, masky_ref, out_ref,
                  acc_ref, dist1_ref):
    b = pl.program_id(0)
    j = pl.program_id(1)

    xm = inp_ref[0] * maskx_ref[0]
    ym = tgt_ref[0] * masky_ref[0]
    xx = jnp.sum(xm * xm, axis=0)
    yy = jnp.sum(ym * ym, axis=0)

    t = jax.lax.dot_general(
        xm, ym, (((0,), (0,)), ((), ())),
        preferred_element_type=jnp.float32)
    t = -2.0 * t

    rowmin = jnp.min(t + yy[None, :], axis=1)
    colmin = jnp.min(t + xx[:, None], axis=0)

    @pl.when((b == 0) & (j == 0))
    def _init():
        acc_ref[0] = 0.0

    acc_ref[0] += jnp.sum(jnp.maximum(colmin + yy, 0.0))

    @pl.when(j == 0)
    def _first():
        dist1_ref[:, 0] = rowmin

    @pl.when(j > 0)
    def _running():
        dist1_ref[:, 0] = jnp.minimum(dist1_ref[:, 0], rowmin)

    @pl.when(j == _J - 1)
    def _finish_rows():
        acc_ref[0] += jnp.sum(jnp.maximum(dist1_ref[:, 0] + xx, 0.0))

    @pl.when((b == _B - 1) & (j == _J - 1))
    def _emit():
        out_ref[0, 0] = acc_ref[0] * (1.0 / (_B * _N))


def kernel(inp, tgt, mask):
    mask3 = mask.reshape(_B, 1, _N)
    out = pl.pallas_call(
        _chamfer_body,
        grid=(_B, _J),
        in_specs=[
            pl.BlockSpec((1, _C, _N), lambda b, j: (b, 0, 0)),
            pl.BlockSpec((1, _C, _TM), lambda b, j: (b, 0, j)),
            pl.BlockSpec((1, 1, _N), lambda b, j: (b, 0, 0)),
            pl.BlockSpec((1, 1, _TM), lambda b, j: (b, 0, j)),
        ],
        out_specs=pl.BlockSpec(
            (1, 1), lambda b, j: (0, 0), memory_space=pltpu.SMEM),
        out_shape=jax.ShapeDtypeStruct((1, 1), jnp.float32),
        scratch_shapes=[
            pltpu.SMEM((1,), jnp.float32),
            pltpu.VMEM((_N, 1), jnp.float32),
        ],
        compiler_params=pltpu.CompilerParams(
            dimension_semantics=("arbitrary", "arbitrary")),
    )(inp, tgt, mask3, mask3)
    return out[0, 0]


# speedup vs baseline: 1.2920x; 1.2920x over previous
import jax
import jax.numpy as jnp
from jax.experimental import pallas as pl
from jax.experimental.pallas import tpu as pltpu

_B, _C, _N = 4, 32, 4096
_TM = 512
_J = _N // _TM
_K = 40


def _chamfer_body(inp_ref, tgt_ref, maskx_ref, masky_ref, out_ref,
                  acc_ref, a_ref, dist1_ref):
    b = pl.program_id(0)
    j = pl.program_id(1)

    @pl.when(j == 0)
    def _start_batch():
        xm = inp_ref[0] * maskx_ref[0]
        xx = jnp.sum(xm * xm, axis=0)
        a_ref[...] = jnp.concatenate(
            [xm, xx[None, :], jnp.ones((1, _N), jnp.float32),
             jnp.zeros((_K - _C - 2, _N), jnp.float32)], axis=0)
        dist1_ref[...] = jnp.full((1, _N), jnp.inf, jnp.float32)

    ym = tgt_ref[0] * masky_ref[0]
    yy = jnp.sum(ym * ym, axis=0)
    bm = jnp.concatenate(
        [-2.0 * ym, jnp.ones((1, _TM), jnp.float32), yy[None, :],
         jnp.zeros((_K - _C - 2, _TM), jnp.float32)], axis=0)
    a = a_ref[...]

    dims = (((0,), (0,)), ((), ()))
    d = jax.lax.dot_general(a, bm, dims,
                            preferred_element_type=jnp.float32)
    dt = jax.lax.dot_general(bm, a, dims,
                             preferred_element_type=jnp.float32)

    colmin = jnp.min(d, axis=0)
    tilemin = jnp.min(dt, axis=0)

    @pl.when((b == 0) & (j == 0))
    def _init():
        acc_ref[0] = 0.0

    acc_ref[0] += jnp.sum(jnp.maximum(colmin, 0.0))
    dist1_ref[0, :] = jnp.minimum(dist1_ref[0, :], tilemin)

    @pl.when(j == _J - 1)
    def _finish_rows():
        acc_ref[0] += jnp.sum(jnp.maximum(dist1_ref[0, :], 0.0))

    @pl.when((b == _B - 1) & (j == _J - 1))
    def _emit():
        out_ref[0, 0] = acc_ref[0] * (1.0 / (_B * _N))


def kernel(inp, tgt, mask):
    mask3 = mask.reshape(_B, 1, _N)
    out = pl.pallas_call(
        _chamfer_body,
        grid=(_B, _J),
        in_specs=[
            pl.BlockSpec((1, _C, _N), lambda b, j: (b, 0, 0)),
            pl.BlockSpec((1, _C, _TM), lambda b, j: (b, 0, j)),
            pl.BlockSpec((1, 1, _N), lambda b, j: (b, 0, 0)),
            pl.BlockSpec((1, 1, _TM), lambda b, j: (b, 0, j)),
        ],
        out_specs=pl.BlockSpec(
            (1, 1), lambda b, j: (0, 0), memory_space=pltpu.SMEM),
        out_shape=jax.ShapeDtypeStruct((1, 1), jnp.float32),
        scratch_shapes=[
            pltpu.SMEM((1,), jnp.float32),
            pltpu.VMEM((_K, _N), jnp.float32),
            pltpu.VMEM((1, _N), jnp.float32),
        ],
        compiler_params=pltpu.CompilerParams(
            dimension_semantics=("arbitrary", "arbitrary")),
    )(inp, tgt, mask3, mask3)
    return out[0, 0]


# speedup vs baseline: 1.9987x; 1.5469x over previous
import jax
import jax.numpy as jnp
from jax.experimental import pallas as pl
from jax.experimental.pallas import tpu as pltpu

_B, _C, _N = 4, 32, 4096
_TM = 512
_J = _N // _TM
_K = 40


def _chamfer_body(inp_ref, tgt_ref, maskx_ref, masky_ref, out_ref,
                  acc_ref, a_ref, dist1_ref):
    b = pl.program_id(0)
    j = pl.program_id(1)

    @pl.when(j == 0)
    def _start_batch():
        xm = inp_ref[0] * maskx_ref[0]
        xx = jnp.sum(xm * xm, axis=0)
        a_ref[...] = jnp.concatenate(
            [xm, xx[None, :], jnp.ones((1, _N), jnp.float32),
             jnp.zeros((_K - _C - 2, _N), jnp.float32)], axis=0)
        dist1_ref[...] = jnp.full((1, _N), jnp.inf, jnp.float32)

    ym = tgt_ref[0] * masky_ref[0]
    yy = jnp.sum(ym * ym, axis=0)
    bm = jnp.concatenate(
        [-2.0 * ym, jnp.ones((1, _TM), jnp.float32), yy[None, :],
         jnp.zeros((_K - _C - 2, _TM), jnp.float32)], axis=0)
    a = a_ref[...]

    dims = (((0,), (0,)), ((), ()))
    dt = jax.lax.dot_general(bm, a, dims,
                             preferred_element_type=jnp.float32)

    colmin = jnp.min(dt, axis=1)
    tilemin = jnp.min(dt, axis=0)

    @pl.when((b == 0) & (j == 0))
    def _init():
        acc_ref[0] = 0.0

    acc_ref[0] += jnp.sum(jnp.maximum(colmin, 0.0))
    dist1_ref[0, :] = jnp.minimum(dist1_ref[0, :], tilemin)

    @pl.when(j == _J - 1)
    def _finish_rows():
        acc_ref[0] += jnp.sum(jnp.maximum(dist1_ref[0, :], 0.0))

    @pl.when((b == _B - 1) & (j == _J - 1))
    def _emit():
        out_ref[0, 0] = acc_ref[0] * (1.0 / (_B * _N))


def kernel(inp, tgt, mask):
    mask3 = mask.reshape(_B, 1, _N)
    out = pl.pallas_call(
        _chamfer_body,
        grid=(_B, _J),
        in_specs=[
            pl.BlockSpec((1, _C, _N), lambda b, j: (b, 0, 0)),
            pl.BlockSpec((1, _C, _TM), lambda b, j: (b, 0, j)),
            pl.BlockSpec((1, 1, _N), lambda b, j: (b, 0, 0)),
            pl.BlockSpec((1, 1, _TM), lambda b, j: (b, 0, j)),
        ],
        out_specs=pl.BlockSpec(
            (1, 1), lambda b, j: (0, 0), memory_space=pltpu.SMEM),
        out_shape=jax.ShapeDtypeStruct((1, 1), jnp.float32),
        scratch_shapes=[
            pltpu.SMEM((1,), jnp.float32),
            pltpu.VMEM((_K, _N), jnp.float32),
            pltpu.VMEM((1, _N), jnp.float32),
        ],
        compiler_params=pltpu.CompilerParams(
            dimension_semantics=("arbitrary", "arbitrary")),
    )(inp, tgt, mask3, mask3)
    return out[0, 0]


# speedup vs baseline: 2.3166x; 1.1591x over previous
import jax
import jax.numpy as jnp
from jax.experimental import pallas as pl
from jax.experimental.pallas import tpu as pltpu

_B, _C, _N = 4, 32, 4096
_TM = 1024
_J = _N // _TM
_K = 40


def _chamfer_body(inp_ref, tgt_ref, maskx_ref, masky_ref, out_ref,
                  acc_ref, a_ref, dist1_ref):
    b = pl.program_id(0)
    j = pl.program_id(1)

    @pl.when(j == 0)
    def _start_batch():
        xm = inp_ref[0] * maskx_ref[0]
        xx = jnp.sum(xm * xm, axis=0)
        a_ref[...] = jnp.concatenate(
            [xm, xx[None, :], jnp.ones((1, _N), jnp.float32),
             jnp.zeros((_K - _C - 2, _N), jnp.float32)], axis=0)
        dist1_ref[...] = jnp.full((1, _N), jnp.inf, jnp.float32)

    ym = tgt_ref[0] * masky_ref[0]
    yy = jnp.sum(ym * ym, axis=0)
    bm = jnp.concatenate(
        [-2.0 * ym, jnp.ones((1, _TM), jnp.float32), yy[None, :],
         jnp.zeros((_K - _C - 2, _TM), jnp.float32)], axis=0)
    a = a_ref[...]

    dims = (((0,), (0,)), ((), ()))
    dt = jax.lax.dot_general(bm, a, dims,
                             preferred_element_type=jnp.float32)

    colmin = jnp.min(dt, axis=1)
    tilemin = jnp.min(dt, axis=0)

    @pl.when((b == 0) & (j == 0))
    def _init():
        acc_ref[0] = 0.0

    acc_ref[0] += jnp.sum(jnp.maximum(colmin, 0.0))
    dist1_ref[0, :] = jnp.minimum(dist1_ref[0, :], tilemin)

    @pl.when(j == _J - 1)
    def _finish_rows():
        acc_ref[0] += jnp.sum(jnp.maximum(dist1_ref[0, :], 0.0))

    @pl.when((b == _B - 1) & (j == _J - 1))
    def _emit():
        out_ref[0, 0] = acc_ref[0] * (1.0 / (_B * _N))


def kernel(inp, tgt, mask):
    mask3 = mask.reshape(_B, 1, _N)
    out = pl.pallas_call(
        _chamfer_body,
        grid=(_B, _J),
        in_specs=[
            pl.BlockSpec((1, _C, _N), lambda b, j: (b, 0, 0)),
            pl.BlockSpec((1, _C, _TM), lambda b, j: (b, 0, j)),
            pl.BlockSpec((1, 1, _N), lambda b, j: (b, 0, 0)),
            pl.BlockSpec((1, 1, _TM), lambda b, j: (b, 0, j)),
        ],
        out_specs=pl.BlockSpec(
            (1, 1), lambda b, j: (0, 0), memory_space=pltpu.SMEM),
        out_shape=jax.ShapeDtypeStruct((1, 1), jnp.float32),
        scratch_shapes=[
            pltpu.SMEM((1,), jnp.float32),
            pltpu.VMEM((_K, _N), jnp.float32),
            pltpu.VMEM((1, _N), jnp.float32),
        ],
        compiler_params=pltpu.CompilerParams(
            dimension_semantics=("arbitrary", "arbitrary")),
    )(inp, tgt, mask3, mask3)
    return out[0, 0]


# speedup vs baseline: 2.5219x; 1.0886x over previous
import jax
import jax.numpy as jnp
from jax.experimental import pallas as pl
from jax.experimental.pallas import tpu as pltpu

_B, _C, _N = 4, 32, 4096
_TM = 2048
_J = _N // _TM
_K = 40


def _chamfer_body(inp_ref, tgt_ref, maskx_ref, masky_ref, out_ref,
                  acc_ref, a_ref, dist1_ref):
    b = pl.program_id(0)
    j = pl.program_id(1)

    @pl.when(j == 0)
    def _start_batch():
        xm = inp_ref[0] * maskx_ref[0]
        xx = jnp.sum(xm * xm, axis=0)
        a_ref[...] = jnp.concatenate(
            [xm, xx[None, :], jnp.ones((1, _N), jnp.float32),
             jnp.zeros((_K - _C - 2, _N), jnp.float32)], axis=0)
        dist1_ref[...] = jnp.full((1, _N), jnp.inf, jnp.float32)

    ym = tgt_ref[0] * masky_ref[0]
    yy = jnp.sum(ym * ym, axis=0)
    bm = jnp.concatenate(
        [-2.0 * ym, jnp.ones((1, _TM), jnp.float32), yy[None, :],
         jnp.zeros((_K - _C - 2, _TM), jnp.float32)], axis=0)
    a = a_ref[...]

    dims = (((0,), (0,)), ((), ()))
    dt = jax.lax.dot_general(bm, a, dims,
                             preferred_element_type=jnp.float32)

    colmin = jnp.min(dt, axis=1)
    tilemin = jnp.min(dt, axis=0)

    @pl.when((b == 0) & (j == 0))
    def _init():
        acc_ref[0] = 0.0

    acc_ref[0] += jnp.sum(jnp.maximum(colmin, 0.0))
    dist1_ref[0, :] = jnp.minimum(dist1_ref[0, :], tilemin)

    @pl.when(j == _J - 1)
    def _finish_rows():
        acc_ref[0] += jnp.sum(jnp.maximum(dist1_ref[0, :], 0.0))

    @pl.when((b == _B - 1) & (j == _J - 1))
    def _emit():
        out_ref[0, 0] = acc_ref[0] * (1.0 / (_B * _N))


def kernel(inp, tgt, mask):
    mask3 = mask.reshape(_B, 1, _N)
    out = pl.pallas_call(
        _chamfer_body,
        grid=(_B, _J),
        in_specs=[
            pl.BlockSpec((1, _C, _N), lambda b, j: (b, 0, 0)),
            pl.BlockSpec((1, _C, _TM), lambda b, j: (b, 0, j)),
            pl.BlockSpec((1, 1, _N), lambda b, j: (b, 0, 0)),
            pl.BlockSpec((1, 1, _TM), lambda b, j: (b, 0, j)),
        ],
        out_specs=pl.BlockSpec(
            (1, 1), lambda b, j: (0, 0), memory_space=pltpu.SMEM),
        out_shape=jax.ShapeDtypeStruct((1, 1), jnp.float32),
        scratch_shapes=[
            pltpu.SMEM((1,), jnp.float32),
            pltpu.VMEM((_K, _N), jnp.float32),
            pltpu.VMEM((1, _N), jnp.float32),
        ],
        compiler_params=pltpu.CompilerParams(
            dimension_semantics=("arbitrary", "arbitrary")),
    )(inp, tgt, mask3, mask3)
    return out[0, 0]
